# baseline (speedup 1.0000x reference)
import functools

import jax
import jax.numpy as jnp
from jax import lax
from jax.experimental import pallas as pl
from jax.experimental.pallas import tpu as pltpu
from jax.experimental.pallas import tpu_sc as plsc

N = 10000
NP = 10240
T = 12
C = 64
E = 160000
NC = 2
NS = 16
NB = 640
GB = NP // NB
NSLICE = NP // NS
ECH = 125
NCH = (E // NS) // ECH
TPC = T // NC
CH = 32
NPASS = TPC * (C // CH)

_f32 = jnp.float32


def _front_body(x_ref, deg_ref, w1_ref, b1_ref, wg_ref, o_ref, dinv_ref):
    xf = x_ref[...].reshape(T * NB, C)
    a0 = jnp.dot(xf, w1_ref[0], preferred_element_type=_f32).reshape(T, NB, C)
    a1 = jnp.dot(xf, w1_ref[1], preferred_element_type=_f32).reshape(T, NB, C)
    a2 = jnp.dot(xf, w1_ref[2], preferred_element_type=_f32).reshape(T, NB, C)
    z = jnp.zeros((1, NB, C), _f32)
    h = a1 + jnp.concatenate([z, a0[:-1]], 0) + jnp.concatenate([a2[1:], z], 0)
    h = jnp.maximum(h + b1_ref[0][None, None, :], 0.0)
    hw = jnp.dot(h.reshape(T * NB, C), wg_ref[...], preferred_element_type=_f32)
    dv = lax.rsqrt(deg_ref[...] + 1.0)
    dinv_ref[...] = dv
    o_ref[...] = hw.reshape(T, NB, C) * dv[None, :, 0:1]


_front = pl.pallas_call(
    _front_body,
    grid=(GB,),
    in_specs=[
        pl.BlockSpec((T, NB, C), lambda i: (0, i, 0)),
        pl.BlockSpec((NB, 16), lambda i: (i, 0)),
        pl.BlockSpec((3, C, C), lambda i: (0, 0, 0)),
        pl.BlockSpec((1, C), lambda i: (0, 0)),
        pl.BlockSpec((C, C), lambda i: (0, 0)),
    ],
    out_specs=[
        pl.BlockSpec((T, NB, C), lambda i: (0, i, 0)),
        pl.BlockSpec((NB, 16), lambda i: (i, 0)),
    ],
    out_shape=[
        jax.ShapeDtypeStruct((T, NP, C), _f32),
        jax.ShapeDtypeStruct((NP, 16), _f32),
    ],
)


def _deg_body(dst_hbm, deg_hbm, dst_v, ones_v, zrow, deg_sp):
    cid = lax.axis_index("c")
    sid = lax.axis_index("s")
    nbase = sid * NSLICE

    @pl.when(cid == 0)
    def _():
        pltpu.sync_copy(dst_hbm.at[sid], dst_v)
        one16 = jnp.ones((16,), _f32)
        zero16 = jnp.zeros((16,), _f32)

        def _fill(j, c):
            ones_v[j] = one16
            return c
        lax.fori_loop(0, ECH, _fill, 0)

        def _zero(n, c):
            zrow[n] = zero16
            return c
        lax.fori_loop(0, NSLICE, _zero, 0)
        pltpu.sync_copy(zrow, deg_sp.at[pl.ds(nbase, NSLICE)])
        plsc.subcore_barrier()

        def _deg_step(i, c):
            pltpu.sync_copy(ones_v, deg_sp.at[dst_v.at[i]], add=True)
            return c
        lax.fori_loop(0, NCH, _deg_step, 0)
        plsc.subcore_barrier()
        pltpu.sync_copy(deg_sp.at[pl.ds(nbase, NSLICE)], zrow)
        pltpu.sync_copy(zrow, deg_hbm.at[pl.ds(nbase, NSLICE)])


@functools.cache
def _sc_deg_fn():
  return pl.kernel(
    _deg_body,
    out_type=jax.ShapeDtypeStruct((NP, 16), _f32),
    mesh=plsc.VectorSubcoreMesh(core_axis_name="c", subcore_axis_name="s",
                                num_cores=NC, num_subcores=NS),
    compiler_params=pltpu.CompilerParams(use_tc_tiling_on_sc=False),
    scratch_types=[
        pltpu.VMEM((NCH, ECH), jnp.int32),
        pltpu.VMEM((ECH, 16), _f32),
        pltpu.VMEM((NSLICE, 16), _f32),
        pltpu.VMEM_SHARED((NP, 16), _f32),
    ],
  )


def _sc_body(src_hbm, dst_hbm, p_hbm, dinv_hbm, s_hbm,
             src_v, dst_v, dv, hrows, grows, grows_b,
             acc_sp, sem, sem_b, sem_sa, sem_sb):
    cid = lax.axis_index("c")
    sid = lax.axis_index("s")
    nbase = sid * NSLICE
    nsl = pl.ds(nbase, NSLICE)

    pltpu.sync_copy(src_hbm.at[sid], src_v)
    pltpu.sync_copy(dst_hbm.at[sid], dst_v)
    pltpu.sync_copy(dinv_hbm.at[nsl], dv)

    def _scale_rows(n, c):
        dn = dv[n]
        for j in range(C // 16):
            sl = pl.ds(j * 16, 16)
            hrows[n, sl] = hrows[n, sl] * dn
        return c

    def _per_t(tt, c):
        t = cid * TPC + tt
        pltpu.sync_copy(p_hbm.at[t, nsl], hrows)
        pltpu.sync_copy(hrows, acc_sp.at[nsl])
        plsc.subcore_barrier()

        pltpu.async_copy(p_hbm.at[t].at[src_v.at[0]], grows, sem)

        def _edge_pair(j, cc):
            i0 = 2 * j
            pltpu.make_async_copy(
                p_hbm.at[t].at[src_v.at[i0]], grows, sem).wait()
            pltpu.async_copy(grows, acc_sp.at[dst_v.at[i0]], sem_sa, add=True)

            @pl.when(j > 0)
            def _():
                pltpu.make_async_copy(
                    grows_b, acc_sp.at[dst_v.at[i0 - 1]], sem_sb).wait()
            pltpu.async_copy(p_hbm.at[t].at[src_v.at[i0 + 1]], grows_b, sem_b)
            pltpu.make_async_copy(
                p_hbm.at[t].at[src_v.at[i0 + 1]], grows_b, sem_b).wait()
            pltpu.async_copy(
                grows_b, acc_sp.at[dst_v.at[i0 + 1]], sem_sb, add=True)
            pltpu.make_async_copy(
                grows, acc_sp.at[dst_v.at[i0]], sem_sa).wait()

            @pl.when(i0 + 2 < NCH)
            def _():
                pltpu.async_copy(p_hbm.at[t].at[src_v.at[i0 + 2]], grows, sem)
            return cc
        lax.fori_loop(0, NCH // 2, _edge_pair, 0)
        pltpu.make_async_copy(
            grows_b, acc_sp.at[dst_v.at[NCH - 1]], sem_sb).wait()
        plsc.subcore_barrier()

        pltpu.sync_copy(acc_sp.at[nsl], hrows)
        lax.fori_loop(0, NSLICE, _scale_rows, 0)
        pltpu.sync_copy(hrows, s_hbm.at[t, nsl])
        return c
    lax.fori_loop(0, TPC, _per_t, 0)


@functools.cache
def _sc_agg_fn():
  return pl.kernel(
    _sc_body,
    out_type=jax.ShapeDtypeStruct((T, NP, C), _f32),
    mesh=plsc.VectorSubcoreMesh(core_axis_name="c", subcore_axis_name="s",
                                num_cores=NC, num_subcores=NS),
    compiler_params=pltpu.CompilerParams(use_tc_tiling_on_sc=False),
    scratch_types=[
        pltpu.VMEM((NCH, ECH), jnp.int32),
        pltpu.VMEM((NCH, ECH), jnp.int32),
        pltpu.VMEM((NSLICE, 16), _f32),
        pltpu.VMEM((NSLICE, C), _f32),
        pltpu.VMEM((ECH, C), _f32),
        pltpu.VMEM((ECH, C), _f32),
        pltpu.VMEM_SHARED((NP, C), _f32),
        pltpu.SemaphoreType.DMA,
        pltpu.SemaphoreType.DMA,
        pltpu.SemaphoreType.DMA,
        pltpu.SemaphoreType.DMA,
    ],
  )


def _mid_body(s_ref, x_ref, bg_ref, lng_ref, lnb_ref, w2_ref, b2_ref,
              y_ref, sums_ref):
    i = pl.program_id(0)
    agg = s_ref[...] + bg_ref[0][None, None, :]
    mu = jnp.mean(agg, axis=-1, keepdims=True)
    cen = agg - mu
    var = jnp.mean(cen * cen, axis=-1, keepdims=True)
    o = cen * lax.rsqrt(var + 1e-5) * lng_ref[0][None, None, :]
    o = jnp.maximum(o + lnb_ref[0][None, None, :], 0.0)
    of = o.reshape(T * NB, C)
    b0 = jnp.dot(of, w2_ref[0], preferred_element_type=_f32).reshape(T, NB, C)
    b1 = jnp.dot(of, w2_ref[1], preferred_element_type=_f32).reshape(T, NB, C)
    b2 = jnp.dot(of, w2_ref[2], preferred_element_type=_f32).reshape(T, NB, C)
    z = jnp.zeros((1, NB, C), _f32)
    y = b1 + jnp.concatenate([z, b0[:-1]], 0) + jnp.concatenate([b2[1:], z], 0)
    y = y + b2_ref[0][None, None, :] + x_ref[...]
    y_ref[...] = y

    rowid = i * NB + lax.broadcasted_iota(jnp.int32, (1, NB, 1), 1)
    ym = jnp.where(rowid < N, y, 0.0)
    s1 = jnp.sum(ym.reshape(T * NB, C), axis=0)
    s2 = jnp.sum((ym * ym).reshape(T * NB, C), axis=0)
    contrib = jnp.stack([s1, s2], 0)

    @pl.when(i == 0)
    def _():
        sums_ref[...] = contrib

    @pl.when(i != 0)
    def _():
        sums_ref[...] = sums_ref[...] + contrib


_mid = pl.pallas_call(
    _mid_body,
    grid=(GB,),
    in_specs=[
        pl.BlockSpec((T, NB, C), lambda i: (0, i, 0)),
        pl.BlockSpec((T, NB, C), lambda i: (0, i, 0)),
        pl.BlockSpec((1, C), lambda i: (0, 0)),
        pl.BlockSpec((1, C), lambda i: (0, 0)),
        pl.BlockSpec((1, C), lambda i: (0, 0)),
        pl.BlockSpec((3, C, C), lambda i: (0, 0, 0)),
        pl.BlockSpec((1, C), lambda i: (0, 0)),
    ],
    out_specs=[
        pl.BlockSpec((T, NB, C), lambda i: (0, i, 0)),
        pl.BlockSpec((2, C), lambda i: (0, 0)),
    ],
    out_shape=[
        jax.ShapeDtypeStruct((T, NP, C), _f32),
        jax.ShapeDtypeStruct((2, C), _f32),
    ],
)


def _bn_body(y_ref, sums_ref, bng_ref, bnb_ref, o_ref):
    cnt = float(N * T)
    m = sums_ref[0, :] / cnt
    v = sums_ref[1, :] / cnt - m * m
    sc = lax.rsqrt(v + 1e-5) * bng_ref[0]
    o_ref[...] = (y_ref[...] - m[None, None, :]) * sc[None, None, :] \
        + bnb_ref[0][None, None, :]


_bn = pl.pallas_call(
    _bn_body,
    grid=(GB,),
    in_specs=[
        pl.BlockSpec((T, NB, C), lambda i: (0, i, 0)),
        pl.BlockSpec((2, C), lambda i: (0, 0)),
        pl.BlockSpec((1, C), lambda i: (0, 0)),
        pl.BlockSpec((1, C), lambda i: (0, 0)),
    ],
    out_specs=pl.BlockSpec((T, NB, C), lambda i: (0, i, 0)),
    out_shape=jax.ShapeDtypeStruct((T, NP, C), _f32),
)


def kernel(x, edge_index, W1, b1, Wg, bg, ln_g, ln_b, W2, b2, bn_g, bn_b):
    xt = jnp.transpose(x[0], (2, 1, 0))
    xt = jnp.pad(xt, ((0, 0), (0, NP - N), (0, 0)))
    w1t = jnp.transpose(W1[:, :, 0, :], (2, 1, 0))
    w2t = jnp.transpose(W2[:, :, 0, :], (2, 1, 0))
    src = edge_index[0].reshape(NS, NCH, ECH)
    dst = edge_index[1].reshape(NS, NCH, ECH)

    deg = _sc_deg_fn()(dst)
    p, dinv = _front(xt, deg, w1t, b1.reshape(1, C), Wg)
    s = _sc_agg_fn()(src, dst, p, dinv)
    y, sums = _mid(s, xt, bg.reshape(1, C), ln_g.reshape(1, C),
                   ln_b.reshape(1, C), w2t, b2.reshape(1, C))
    o = _bn(y, sums, bn_g.reshape(1, C), bn_b.reshape(1, C))
    return jnp.transpose(o[:, :N, :], (2, 1, 0))[None]


# speedup vs baseline: 1.0084x; 1.0084x over previous
import functools

import jax
import jax.numpy as jnp
from jax import lax
from jax.experimental import pallas as pl
from jax.experimental.pallas import tpu as pltpu
from jax.experimental.pallas import tpu_sc as plsc

N = 10000
NP = 10240
T = 12
C = 64
E = 160000
NC = 2
NS = 16
NB = 1280
GB = NP // NB
NSLICE = NP // NS
ECH = 125
NCH = (E // NS) // ECH
TPC = T // NC
CH = 32
NPASS = TPC * (C // CH)

_f32 = jnp.float32


def _front_body(x_ref, deg_ref, w1_ref, b1_ref, wg_ref, o_ref, dinv_ref):
    xf = x_ref[...].reshape(T * NB, C)
    a0 = jnp.dot(xf, w1_ref[0], preferred_element_type=_f32).reshape(T, NB, C)
    a1 = jnp.dot(xf, w1_ref[1], preferred_element_type=_f32).reshape(T, NB, C)
    a2 = jnp.dot(xf, w1_ref[2], preferred_element_type=_f32).reshape(T, NB, C)
    z = jnp.zeros((1, NB, C), _f32)
    h = a1 + jnp.concatenate([z, a0[:-1]], 0) + jnp.concatenate([a2[1:], z], 0)
    h = jnp.maximum(h + b1_ref[0][None, None, :], 0.0)
    hw = jnp.dot(h.reshape(T * NB, C), wg_ref[...], preferred_element_type=_f32)
    dv = lax.rsqrt(deg_ref[...] + 1.0)
    dinv_ref[...] = dv
    o_ref[...] = hw.reshape(T, NB, C) * dv[None, :, 0:1]


_front = pl.pallas_call(
    _front_body,
    grid=(GB,),
    in_specs=[
        pl.BlockSpec((T, NB, C), lambda i: (0, i, 0)),
        pl.BlockSpec((NB, 16), lambda i: (i, 0)),
        pl.BlockSpec((3, C, C), lambda i: (0, 0, 0)),
        pl.BlockSpec((1, C), lambda i: (0, 0)),
        pl.BlockSpec((C, C), lambda i: (0, 0)),
    ],
    out_specs=[
        pl.BlockSpec((T, NB, C), lambda i: (0, i, 0)),
        pl.BlockSpec((NB, 16), lambda i: (i, 0)),
    ],
    out_shape=[
        jax.ShapeDtypeStruct((T, NP, C), _f32),
        jax.ShapeDtypeStruct((NP, 16), _f32),
    ],
)


def _deg_body(dst_hbm, deg_hbm, dst_v, ones_v, zrow, deg_sp):
    cid = lax.axis_index("c")
    sid = lax.axis_index("s")
    nbase = sid * NSLICE

    @pl.when(cid == 0)
    def _():
        pltpu.sync_copy(dst_hbm.at[sid], dst_v)
        one16 = jnp.ones((16,), _f32)
        zero16 = jnp.zeros((16,), _f32)

        def _fill(j, c):
            ones_v[j] = one16
            return c
        lax.fori_loop(0, ECH, _fill, 0)

        def _zero(n, c):
            zrow[n] = zero16
            return c
        lax.fori_loop(0, NSLICE, _zero, 0)
        pltpu.sync_copy(zrow, deg_sp.at[pl.ds(nbase, NSLICE)])
        plsc.subcore_barrier()

        def _deg_step(i, c):
            pltpu.sync_copy(ones_v, deg_sp.at[dst_v.at[i]], add=True)
            return c
        lax.fori_loop(0, NCH, _deg_step, 0)
        plsc.subcore_barrier()
        pltpu.sync_copy(deg_sp.at[pl.ds(nbase, NSLICE)], zrow)
        pltpu.sync_copy(zrow, deg_hbm.at[pl.ds(nbase, NSLICE)])


@functools.cache
def _sc_deg_fn():
  return pl.kernel(
    _deg_body,
    out_type=jax.ShapeDtypeStruct((NP, 16), _f32),
    mesh=plsc.VectorSubcoreMesh(core_axis_name="c", subcore_axis_name="s",
                                num_cores=NC, num_subcores=NS),
    compiler_params=pltpu.CompilerParams(use_tc_tiling_on_sc=False),
    scratch_types=[
        pltpu.VMEM((NCH, ECH), jnp.int32),
        pltpu.VMEM((ECH, 16), _f32),
        pltpu.VMEM((NSLICE, 16), _f32),
        pltpu.VMEM_SHARED((NP, 16), _f32),
    ],
  )


def _sc_body(src_hbm, dst_hbm, p_hbm, dinv_hbm, s_hbm,
             src_v, dst_v, dv, hrows, grows, grows_b,
             acc_sp, sem, sem_b, sem_sa, sem_sb):
    cid = lax.axis_index("c")
    sid = lax.axis_index("s")
    nbase = sid * NSLICE
    nsl = pl.ds(nbase, NSLICE)

    pltpu.sync_copy(src_hbm.at[sid], src_v)
    pltpu.sync_copy(dst_hbm.at[sid], dst_v)
    pltpu.sync_copy(dinv_hbm.at[nsl], dv)

    def _scale_rows(n, c):
        dn = dv[n]
        for j in range(C // 16):
            sl = pl.ds(j * 16, 16)
            hrows[n, sl] = hrows[n, sl] * dn
        return c

    def _per_t(tt, c):
        t = cid * TPC + tt
        pltpu.sync_copy(p_hbm.at[t, nsl], hrows)
        pltpu.sync_copy(hrows, acc_sp.at[nsl])
        plsc.subcore_barrier()

        pltpu.async_copy(p_hbm.at[t].at[src_v.at[0]], grows, sem)

        def _edge_pair(j, cc):
            i0 = 2 * j
            pltpu.make_async_copy(
                p_hbm.at[t].at[src_v.at[i0]], grows, sem).wait()
            pltpu.async_copy(grows, acc_sp.at[dst_v.at[i0]], sem_sa, add=True)

            @pl.when(j > 0)
            def _():
                pltpu.make_async_copy(
                    grows_b, acc_sp.at[dst_v.at[i0 - 1]], sem_sb).wait()
            pltpu.async_copy(p_hbm.at[t].at[src_v.at[i0 + 1]], grows_b, sem_b)
            pltpu.make_async_copy(
                p_hbm.at[t].at[src_v.at[i0 + 1]], grows_b, sem_b).wait()
            pltpu.async_copy(
                grows_b, acc_sp.at[dst_v.at[i0 + 1]], sem_sb, add=True)
            pltpu.make_async_copy(
                grows, acc_sp.at[dst_v.at[i0]], sem_sa).wait()

            @pl.when(i0 + 2 < NCH)
            def _():
                pltpu.async_copy(p_hbm.at[t].at[src_v.at[i0 + 2]], grows, sem)
            return cc
        lax.fori_loop(0, NCH // 2, _edge_pair, 0)
        pltpu.make_async_copy(
            grows_b, acc_sp.at[dst_v.at[NCH - 1]], sem_sb).wait()
        plsc.subcore_barrier()

        pltpu.sync_copy(acc_sp.at[nsl], hrows)
        lax.fori_loop(0, NSLICE, _scale_rows, 0)
        pltpu.sync_copy(hrows, s_hbm.at[t, nsl])
        return c
    lax.fori_loop(0, TPC, _per_t, 0)


@functools.cache
def _sc_agg_fn():
  return pl.kernel(
    _sc_body,
    out_type=jax.ShapeDtypeStruct((T, NP, C), _f32),
    mesh=plsc.VectorSubcoreMesh(core_axis_name="c", subcore_axis_name="s",
                                num_cores=NC, num_subcores=NS),
    compiler_params=pltpu.CompilerParams(use_tc_tiling_on_sc=False),
    scratch_types=[
        pltpu.VMEM((NCH, ECH), jnp.int32),
        pltpu.VMEM((NCH, ECH), jnp.int32),
        pltpu.VMEM((NSLICE, 16), _f32),
        pltpu.VMEM((NSLICE, C), _f32),
        pltpu.VMEM((ECH, C), _f32),
        pltpu.VMEM((ECH, C), _f32),
        pltpu.VMEM_SHARED((NP, C), _f32),
        pltpu.SemaphoreType.DMA,
        pltpu.SemaphoreType.DMA,
        pltpu.SemaphoreType.DMA,
        pltpu.SemaphoreType.DMA,
    ],
  )


def _mid_body(s_ref, x_ref, bg_ref, lng_ref, lnb_ref, w2_ref, b2_ref,
              y_ref, sums_ref):
    i = pl.program_id(0)
    agg = s_ref[...] + bg_ref[0][None, None, :]
    mu = jnp.mean(agg, axis=-1, keepdims=True)
    cen = agg - mu
    var = jnp.mean(cen * cen, axis=-1, keepdims=True)
    o = cen * lax.rsqrt(var + 1e-5) * lng_ref[0][None, None, :]
    o = jnp.maximum(o + lnb_ref[0][None, None, :], 0.0)
    of = o.reshape(T * NB, C)
    b0 = jnp.dot(of, w2_ref[0], preferred_element_type=_f32).reshape(T, NB, C)
    b1 = jnp.dot(of, w2_ref[1], preferred_element_type=_f32).reshape(T, NB, C)
    b2 = jnp.dot(of, w2_ref[2], preferred_element_type=_f32).reshape(T, NB, C)
    z = jnp.zeros((1, NB, C), _f32)
    y = b1 + jnp.concatenate([z, b0[:-1]], 0) + jnp.concatenate([b2[1:], z], 0)
    y = y + b2_ref[0][None, None, :] + x_ref[...]
    y_ref[...] = y

    rowid = i * NB + lax.broadcasted_iota(jnp.int32, (1, NB, 1), 1)
    ym = jnp.where(rowid < N, y, 0.0)
    s1 = jnp.sum(ym.reshape(T * NB, C), axis=0)
    s2 = jnp.sum((ym * ym).reshape(T * NB, C), axis=0)
    contrib = jnp.stack([s1, s2], 0)

    @pl.when(i == 0)
    def _():
        sums_ref[...] = contrib

    @pl.when(i != 0)
    def _():
        sums_ref[...] = sums_ref[...] + contrib


_mid = pl.pallas_call(
    _mid_body,
    grid=(GB,),
    in_specs=[
        pl.BlockSpec((T, NB, C), lambda i: (0, i, 0)),
        pl.BlockSpec((T, NB, C), lambda i: (0, i, 0)),
        pl.BlockSpec((1, C), lambda i: (0, 0)),
        pl.BlockSpec((1, C), lambda i: (0, 0)),
        pl.BlockSpec((1, C), lambda i: (0, 0)),
        pl.BlockSpec((3, C, C), lambda i: (0, 0, 0)),
        pl.BlockSpec((1, C), lambda i: (0, 0)),
    ],
    out_specs=[
        pl.BlockSpec((T, NB, C), lambda i: (0, i, 0)),
        pl.BlockSpec((2, C), lambda i: (0, 0)),
    ],
    out_shape=[
        jax.ShapeDtypeStruct((T, NP, C), _f32),
        jax.ShapeDtypeStruct((2, C), _f32),
    ],
)


def _bn_body(y_ref, sums_ref, bng_ref, bnb_ref, o_ref):
    cnt = float(N * T)
    m = sums_ref[0, :] / cnt
    v = sums_ref[1, :] / cnt - m * m
    sc = lax.rsqrt(v + 1e-5) * bng_ref[0]
    o_ref[...] = (y_ref[...] - m[None, None, :]) * sc[None, None, :] \
        + bnb_ref[0][None, None, :]


_bn = pl.pallas_call(
    _bn_body,
    grid=(GB,),
    in_specs=[
        pl.BlockSpec((T, NB, C), lambda i: (0, i, 0)),
        pl.BlockSpec((2, C), lambda i: (0, 0)),
        pl.BlockSpec((1, C), lambda i: (0, 0)),
        pl.BlockSpec((1, C), lambda i: (0, 0)),
    ],
    out_specs=pl.BlockSpec((T, NB, C), lambda i: (0, i, 0)),
    out_shape=jax.ShapeDtypeStruct((T, NP, C), _f32),
)


def kernel(x, edge_index, W1, b1, Wg, bg, ln_g, ln_b, W2, b2, bn_g, bn_b):
    xt = jnp.transpose(x[0], (2, 1, 0))
    xt = jnp.pad(xt, ((0, 0), (0, NP - N), (0, 0)))
    w1t = jnp.transpose(W1[:, :, 0, :], (2, 1, 0))
    w2t = jnp.transpose(W2[:, :, 0, :], (2, 1, 0))
    src = edge_index[0].reshape(NS, NCH, ECH)
    dst = edge_index[1].reshape(NS, NCH, ECH)

    deg = _sc_deg_fn()(dst)
    p, dinv = _front(xt, deg, w1t, b1.reshape(1, C), Wg)
    s = _sc_agg_fn()(src, dst, p, dinv)
    y, sums = _mid(s, xt, bg.reshape(1, C), ln_g.reshape(1, C),
                   ln_b.reshape(1, C), w2t, b2.reshape(1, C))
    o = _bn(y, sums, bn_g.reshape(1, C), bn_b.reshape(1, C))
    return jnp.transpose(o[:, :N, :], (2, 1, 0))[None]


# speedup vs baseline: 1.0211x; 1.0126x over previous
import functools

import jax
import jax.numpy as jnp
from jax import lax
from jax.experimental import pallas as pl
from jax.experimental.pallas import tpu as pltpu
from jax.experimental.pallas import tpu_sc as plsc

N = 10000
NP = 10240
T = 12
C = 64
E = 160000
NC = 2
NS = 16
NB = 1280
GB = NP // NB
NSLICE = NP // NS
ECH = 125
NCH = (E // NS) // ECH
TPC = T // NC
CH = 32
NPASS = TPC * (C // CH)

_f32 = jnp.float32


def _front_body(x_ref, deg_ref, w1_ref, b1_ref, wg_ref, o_ref, dinv_ref):
    xf = x_ref[...].reshape(T * NB, C)
    a0 = jnp.dot(xf, w1_ref[0], preferred_element_type=_f32).reshape(T, NB, C)
    a1 = jnp.dot(xf, w1_ref[1], preferred_element_type=_f32).reshape(T, NB, C)
    a2 = jnp.dot(xf, w1_ref[2], preferred_element_type=_f32).reshape(T, NB, C)
    z = jnp.zeros((1, NB, C), _f32)
    h = a1 + jnp.concatenate([z, a0[:-1]], 0) + jnp.concatenate([a2[1:], z], 0)
    h = jnp.maximum(h + b1_ref[0][None, None, :], 0.0)
    hw = jnp.dot(h.reshape(T * NB, C), wg_ref[...], preferred_element_type=_f32)
    dv = lax.rsqrt(deg_ref[...] + 1.0)
    dinv_ref[...] = dv
    o_ref[...] = hw.reshape(T, NB, C) * dv[None, :, 0:1]


_front = pl.pallas_call(
    _front_body,
    grid=(GB,),
    in_specs=[
        pl.BlockSpec((T, NB, C), lambda i: (0, i, 0)),
        pl.BlockSpec((NB, 16), lambda i: (i, 0)),
        pl.BlockSpec((3, C, C), lambda i: (0, 0, 0)),
        pl.BlockSpec((1, C), lambda i: (0, 0)),
        pl.BlockSpec((C, C), lambda i: (0, 0)),
    ],
    out_specs=[
        pl.BlockSpec((T, NB, C), lambda i: (0, i, 0)),
        pl.BlockSpec((NB, 16), lambda i: (i, 0)),
    ],
    out_shape=[
        jax.ShapeDtypeStruct((T, NP, C), _f32),
        jax.ShapeDtypeStruct((NP, 16), _f32),
    ],
)


def _deg_body(dst_hbm, deg_hbm, dst_v, ones_v, zrow, deg_sp):
    cid = lax.axis_index("c")
    sid = lax.axis_index("s")
    nbase = sid * NSLICE

    @pl.when(cid == 0)
    def _():
        pltpu.sync_copy(dst_hbm.at[sid], dst_v)
        one16 = jnp.ones((16,), _f32)
        zero16 = jnp.zeros((16,), _f32)

        def _fill(j, c):
            ones_v[j] = one16
            return c
        lax.fori_loop(0, ECH, _fill, 0)

        def _zero(n, c):
            zrow[n] = zero16
            return c
        lax.fori_loop(0, NSLICE, _zero, 0)
        pltpu.sync_copy(zrow, deg_sp.at[pl.ds(nbase, NSLICE)])
        plsc.subcore_barrier()

        def _deg_step(i, c):
            pltpu.sync_copy(ones_v, deg_sp.at[dst_v.at[i]], add=True)
            return c
        lax.fori_loop(0, NCH, _deg_step, 0)
        plsc.subcore_barrier()
        pltpu.sync_copy(deg_sp.at[pl.ds(nbase, NSLICE)], zrow)
        pltpu.sync_copy(zrow, deg_hbm.at[pl.ds(nbase, NSLICE)])


@functools.cache
def _sc_deg_fn():
  return pl.kernel(
    _deg_body,
    out_type=jax.ShapeDtypeStruct((NP, 16), _f32),
    mesh=plsc.VectorSubcoreMesh(core_axis_name="c", subcore_axis_name="s",
                                num_cores=NC, num_subcores=NS),
    compiler_params=pltpu.CompilerParams(use_tc_tiling_on_sc=False),
    scratch_types=[
        pltpu.VMEM((NCH, ECH), jnp.int32),
        pltpu.VMEM((ECH, 16), _f32),
        pltpu.VMEM((NSLICE, 16), _f32),
        pltpu.VMEM_SHARED((NP, 16), _f32),
    ],
  )


def _sc_body(src_hbm, dst_hbm, p_hbm, dinv_hbm, s_hbm,
             src_v, dst_v, dv, hrows, grows, grows_b,
             acc_sp, sem, sem_b, sem_sa, sem_sb):
    cid = lax.axis_index("c")
    sid = lax.axis_index("s")
    nbase = sid * NSLICE
    nsl = pl.ds(nbase, NSLICE)

    pltpu.sync_copy(src_hbm.at[sid], src_v)
    pltpu.sync_copy(dst_hbm.at[sid], dst_v)
    pltpu.sync_copy(dinv_hbm.at[nsl], dv)

    def _scale_rows(n, c):
        dn = dv[n]
        for j in range(C // 16):
            sl = pl.ds(j * 16, 16)
            hrows[n, sl] = hrows[n, sl] * dn
        return c

    def _per_t(tt, c):
        t = cid * TPC + tt
        pltpu.sync_copy(p_hbm.at[t, nsl], hrows)
        pltpu.sync_copy(hrows, acc_sp.at[nsl])
        plsc.subcore_barrier()

        pltpu.async_copy(p_hbm.at[t].at[src_v.at[0]], grows, sem)

        def _edge_pair(j, cc):
            i0 = 2 * j
            pltpu.make_async_copy(
                p_hbm.at[t].at[src_v.at[i0]], grows, sem).wait()
            pltpu.async_copy(grows, acc_sp.at[dst_v.at[i0]], sem_sa, add=True)

            @pl.when(j > 0)
            def _():
                pltpu.make_async_copy(
                    grows_b, acc_sp.at[dst_v.at[i0 - 1]], sem_sb).wait()
            pltpu.async_copy(p_hbm.at[t].at[src_v.at[i0 + 1]], grows_b, sem_b)
            pltpu.make_async_copy(
                p_hbm.at[t].at[src_v.at[i0 + 1]], grows_b, sem_b).wait()
            pltpu.async_copy(
                grows_b, acc_sp.at[dst_v.at[i0 + 1]], sem_sb, add=True)
            pltpu.make_async_copy(
                grows, acc_sp.at[dst_v.at[i0]], sem_sa).wait()

            @pl.when(i0 + 2 < NCH)
            def _():
                pltpu.async_copy(p_hbm.at[t].at[src_v.at[i0 + 2]], grows, sem)
            return cc
        lax.fori_loop(0, NCH // 2, _edge_pair, 0)
        pltpu.make_async_copy(
            grows_b, acc_sp.at[dst_v.at[NCH - 1]], sem_sb).wait()
        plsc.subcore_barrier()

        pltpu.sync_copy(acc_sp.at[nsl], hrows)
        lax.fori_loop(0, NSLICE, _scale_rows, 0)
        pltpu.sync_copy(hrows, s_hbm.at[t, nsl])
        return c
    lax.fori_loop(0, TPC, _per_t, 0)


@functools.cache
def _sc_agg_fn():
  return pl.kernel(
    _sc_body,
    out_type=jax.ShapeDtypeStruct((T, NP, C), _f32),
    mesh=plsc.VectorSubcoreMesh(core_axis_name="c", subcore_axis_name="s",
                                num_cores=NC, num_subcores=NS),
    compiler_params=pltpu.CompilerParams(use_tc_tiling_on_sc=False),
    scratch_types=[
        pltpu.VMEM((NCH, ECH), jnp.int32),
        pltpu.VMEM((NCH, ECH), jnp.int32),
        pltpu.VMEM((NSLICE, 16), _f32),
        pltpu.VMEM((NSLICE, C), _f32),
        pltpu.VMEM((ECH, C), _f32),
        pltpu.VMEM((ECH, C), _f32),
        pltpu.VMEM_SHARED((NP, C), _f32),
        pltpu.SemaphoreType.DMA,
        pltpu.SemaphoreType.DMA,
        pltpu.SemaphoreType.DMA,
        pltpu.SemaphoreType.DMA,
    ],
  )


def _mid_body(s_ref, x_ref, bg_ref, lng_ref, lnb_ref, w2_ref, b2_ref,
              y_ref, sums_ref):
    i = pl.program_id(0)
    agg = s_ref[...] + bg_ref[0][None, None, :]
    mu = jnp.mean(agg, axis=-1, keepdims=True)
    cen = agg - mu
    var = jnp.mean(cen * cen, axis=-1, keepdims=True)
    o = cen * lax.rsqrt(var + 1e-5) * lng_ref[0][None, None, :]
    o = jnp.maximum(o + lnb_ref[0][None, None, :], 0.0)
    of = o.reshape(T * NB, C)
    b0 = jnp.dot(of, w2_ref[0], preferred_element_type=_f32).reshape(T, NB, C)
    b1 = jnp.dot(of, w2_ref[1], preferred_element_type=_f32).reshape(T, NB, C)
    b2 = jnp.dot(of, w2_ref[2], preferred_element_type=_f32).reshape(T, NB, C)
    z = jnp.zeros((1, NB, C), _f32)
    y = b1 + jnp.concatenate([z, b0[:-1]], 0) + jnp.concatenate([b2[1:], z], 0)
    y = y + b2_ref[0][None, None, :] + x_ref[...]
    y_ref[...] = y.astype(jnp.bfloat16)

    rowid = i * NB + lax.broadcasted_iota(jnp.int32, (1, NB, 1), 1)
    ym = jnp.where(rowid < N, y, 0.0)
    s1 = jnp.sum(ym.reshape(T * NB, C), axis=0)
    s2 = jnp.sum((ym * ym).reshape(T * NB, C), axis=0)
    contrib = jnp.stack([s1, s2], 0)

    @pl.when(i == 0)
    def _():
        sums_ref[...] = contrib

    @pl.when(i != 0)
    def _():
        sums_ref[...] = sums_ref[...] + contrib


_mid = pl.pallas_call(
    _mid_body,
    grid=(GB,),
    in_specs=[
        pl.BlockSpec((T, NB, C), lambda i: (0, i, 0)),
        pl.BlockSpec((T, NB, C), lambda i: (0, i, 0)),
        pl.BlockSpec((1, C), lambda i: (0, 0)),
        pl.BlockSpec((1, C), lambda i: (0, 0)),
        pl.BlockSpec((1, C), lambda i: (0, 0)),
        pl.BlockSpec((3, C, C), lambda i: (0, 0, 0)),
        pl.BlockSpec((1, C), lambda i: (0, 0)),
    ],
    out_specs=[
        pl.BlockSpec((T, NB, C), lambda i: (0, i, 0)),
        pl.BlockSpec((2, C), lambda i: (0, 0)),
    ],
    out_shape=[
        jax.ShapeDtypeStruct((T, NP, C), jnp.bfloat16),
        jax.ShapeDtypeStruct((2, C), _f32),
    ],
)


def _bn_body(y_ref, sums_ref, bng_ref, bnb_ref, o_ref):
    cnt = float(N * T)
    m = sums_ref[0, :] / cnt
    v = sums_ref[1, :] / cnt - m * m
    sc = lax.rsqrt(v + 1e-5) * bng_ref[0]
    yv = y_ref[...].astype(_f32)
    o_ref[...] = (yv - m[None, None, :]) * sc[None, None, :] \
        + bnb_ref[0][None, None, :]


_bn = pl.pallas_call(
    _bn_body,
    grid=(GB,),
    in_specs=[
        pl.BlockSpec((T, NB, C), lambda i: (0, i, 0)),
        pl.BlockSpec((2, C), lambda i: (0, 0)),
        pl.BlockSpec((1, C), lambda i: (0, 0)),
        pl.BlockSpec((1, C), lambda i: (0, 0)),
    ],
    out_specs=pl.BlockSpec((T, NB, C), lambda i: (0, i, 0)),
    out_shape=jax.ShapeDtypeStruct((T, NP, C), _f32),
)


def kernel(x, edge_index, W1, b1, Wg, bg, ln_g, ln_b, W2, b2, bn_g, bn_b):
    xt = jnp.transpose(x[0], (2, 1, 0))
    xt = jnp.pad(xt, ((0, 0), (0, NP - N), (0, 0)))
    w1t = jnp.transpose(W1[:, :, 0, :], (2, 1, 0))
    w2t = jnp.transpose(W2[:, :, 0, :], (2, 1, 0))
    src = edge_index[0].reshape(NS, NCH, ECH)
    dst = edge_index[1].reshape(NS, NCH, ECH)

    deg = _sc_deg_fn()(dst)
    p, dinv = _front(xt, deg, w1t, b1.reshape(1, C), Wg)
    s = _sc_agg_fn()(src, dst, p, dinv)
    y, sums = _mid(s, xt, bg.reshape(1, C), ln_g.reshape(1, C),
                   ln_b.reshape(1, C), w2t, b2.reshape(1, C))
    o = _bn(y, sums, bn_g.reshape(1, C), bn_b.reshape(1, C))
    return jnp.transpose(o[:, :N, :], (2, 1, 0))[None]
